# baseline (speedup 1.0000x reference)
import functools

import jax
import jax.numpy as jnp
from jax import lax
from jax.experimental import pallas as pl
from jax.experimental.pallas import tpu as pltpu
from jax.experimental.pallas import tpu_sc as plsc

HIDDEN = 2048
EPS = 1e-6
L = 16
NC = 2
NS = 16
NW = NC * NS
CH = 8
NSL = HIDDEN // L


def _rsqrt_newton(x):
    i = lax.bitcast_convert_type(x, jnp.int32)
    i = jnp.full((L,), 0x5F3759DF, jnp.int32) - (i >> 1)
    y = lax.bitcast_convert_type(i, jnp.float32)
    for _ in range(3):
        y = y * (1.5 - 0.5 * x * y * y)
    return y


def _lane_allreduce_sum(x):
    idx = lax.iota(jnp.int32, L)
    for k in (1, 2, 4, 8):
        x = x + x.at[idx ^ k].get(mode="promise_in_bounds")
    return x


def _make_sc_kernel(nb, seq):
    n_tokens = nb * seq
    tpw = n_tokens // NW
    ppw = tpw // nb
    pch = CH // nb
    nch = tpw // CH
    npair = nch // 2

    mesh = plsc.VectorSubcoreMesh(core_axis_name="c", subcore_axis_name="s")

    @functools.partial(
        pl.kernel,
        out_type=jax.ShapeDtypeStruct((n_tokens, HIDDEN), jnp.float32),
        mesh=mesh,
        scratch_types=[
            pltpu.VMEM((tpw,), jnp.int32),
            pltpu.VMEM((CH, HIDDEN), jnp.float32),
            pltpu.VMEM((CH, HIDDEN), jnp.float32),
            pltpu.VMEM((CH, HIDDEN), jnp.float32),
            pltpu.VMEM((CH, HIDDEN), jnp.float32),
            pltpu.VMEM((pch, HIDDEN), jnp.float32),
            pltpu.VMEM((pch, HIDDEN), jnp.float32),
            pltpu.VMEM((pch, HIDDEN), jnp.float32),
            pltpu.VMEM((pch, HIDDEN), jnp.float32),
            pltpu.VMEM((CH, HIDDEN), jnp.float32),
            pltpu.VMEM((HIDDEN,), jnp.float32),
            pltpu.VMEM((HIDDEN,), jnp.float32),
            pltpu.SemaphoreType.DMA,
            pltpu.SemaphoreType.DMA,
            pltpu.SemaphoreType.DMA,
            pltpu.SemaphoreType.DMA,
            pltpu.SemaphoreType.DMA,
            pltpu.SemaphoreType.DMA,
            pltpu.SemaphoreType.DMA,
            pltpu.SemaphoreType.DMA,
            pltpu.SemaphoreType.DMA,
            pltpu.SemaphoreType.DMA,
            pltpu.SemaphoreType.DMA,
            pltpu.SemaphoreType.DMA,
        ],
    )
    def k(ids_hbm, wemb_hbm, pemb_hbm, gamma_hbm, beta_hbm, out_hbm,
          idx_v, rows0, rows1, rows2, rows3, pos0, pos1, pos2, pos3,
          xbuf, g_v, b_v,
          gs0, gs1, gs2, gs3, ps0, ps1, ps2, ps3, ss0, ss1, ss2, ss3):
        cid = lax.axis_index("c")
        sid = lax.axis_index("s")
        wid = sid * NC + cid
        base = wid * tpw
        pos_base = wid * ppw
        pltpu.sync_copy(ids_hbm.at[pl.ds(base, tpw)], idx_v)

        rows = (rows0, rows1, rows2, rows3)
        pos = (pos0, pos1, pos2, pos3)
        gs = (gs0, gs1, gs2, gs3)
        ps = (ps0, ps1, ps2, ps3)
        ss = (ss0, ss1, ss2, ss3)

        def issue(c, b):
            pltpu.async_copy(
                pemb_hbm.at[pl.ds(pos_base + c * pch, pch)], pos[b], ps[b])
            pltpu.async_copy(
                wemb_hbm.at[idx_v.at[pl.ds(c * CH, CH)]], rows[b], gs[b])

        def wait_in(b):
            pltpu.make_async_copy(
                pemb_hbm.at[pl.ds(0, pch)], pos[b], ps[b]).wait()
            pltpu.make_async_copy(
                pemb_hbm.at[pl.ds(0, CH)], rows[b], gs[b]).wait()

        def wait_store(b):
            for bi in range(nb):
                pltpu.make_async_copy(
                    rows[b].at[pl.ds(bi * pch, pch)],
                    out_hbm.at[pl.ds(0, pch)], ss[b]).wait()

        def compute(c, b):
            rv, pv, ov = rows[b], pos[b], rows[b]

            def p1(j, carry):
                accs, ws, ps_ = carry
                sl = pl.ds(j * L, L)
                out = []
                for t in range(CH):
                    x = ws[t] + ps_[t // nb]
                    xbuf[t, sl] = x
                    out.append(accs[2 * t] + x)
                    out.append(accs[2 * t + 1] + x * x)
                jn = (j + 1) & (NSL - 1)
                sln = pl.ds(jn * L, L)
                nws = tuple(rv[t, sln] for t in range(CH))
                nps = tuple(pv[p_, sln] for p_ in range(pch))
                return (tuple(out), nws, nps)

            zero = jnp.zeros((L,), jnp.float32)
            sl0 = pl.ds(0, L)
            w0 = tuple(rv[t, sl0] for t in range(CH))
            p0 = tuple(pv[p_, sl0] for p_ in range(pch))
            st, _, _ = plsc.parallel_loop(
                0, NSL, carry=((zero,) * (2 * CH), w0, p0))(p1)

            stats = []
            for t in range(CH):
                mean = _lane_allreduce_sum(st[2 * t]) * (1.0 / HIDDEN)
                var = (_lane_allreduce_sum(st[2 * t + 1]) * (1.0 / HIDDEN)
                       - mean * mean)
                stats.append((mean, _rsqrt_newton(var + EPS)))

            def p2(j):
                sl = pl.ds(j * L, L)
                g = g_v[sl]
                b_ = b_v[sl]
                for t in range(CH):
                    mean, inv = stats[t]
                    ov[(t % nb) * pch + t // nb, sl] = (
                        (xbuf[t, sl] - mean) * inv * g + b_)

            plsc.parallel_loop(0, NSL, unroll=2)(p2)
            for bi in range(nb):
                pltpu.async_copy(
                    ov.at[pl.ds(bi * pch, pch)],
                    out_hbm.at[pl.ds(bi * seq + pos_base + c * pch, pch)],
                    ss[b])

        issue(0, 0)
        issue(1, 1)
        issue(2, 2)
        pltpu.async_copy(gamma_hbm, g_v, ss0)
        pltpu.async_copy(beta_hbm, b_v, ss1)
        pltpu.make_async_copy(gamma_hbm, g_v, ss0).wait()
        pltpu.make_async_copy(beta_hbm, b_v, ss1).wait()

        def quad(i, _):
            c0 = 4 * i
            for u in range(4):
                c = c0 + u
                wait_in(u)
                compute(c, u)

                @pl.when(c + 3 < nch)
                def _():
                    if u == 0:
                        @pl.when(c >= 1)
                        def _():
                            wait_store(3)
                    else:
                        wait_store((u + 3) % 4)
                    issue(c + 3, (u + 3) % 4)
            return 0

        lax.fori_loop(0, nch // 4, quad, 0)
        for u in range(4):
            wait_store(u)

    return k


def kernel(input_ids, word_emb, pos_emb, gamma, beta):
    b, s = input_ids.shape
    n = b * s
    ids = input_ids.T.reshape(n).astype(jnp.int32)
    k = _make_sc_kernel(b, s)
    out = k(ids, word_emb, pos_emb, gamma, beta)
    return out.reshape(b, s, HIDDEN)


# speedup vs baseline: 1.0356x; 1.0356x over previous
import functools

import jax
import jax.numpy as jnp
from jax import lax
from jax.experimental import pallas as pl
from jax.experimental.pallas import tpu as pltpu
from jax.experimental.pallas import tpu_sc as plsc

HIDDEN = 2048
EPS = 1e-6
L = 16
NC = 2
NS = 16
NW = NC * NS
CH = 8
NSL = HIDDEN // L


def _rsqrt_newton(x):
    i = lax.bitcast_convert_type(x, jnp.int32)
    i = jnp.full((L,), 0x5F3759DF, jnp.int32) - (i >> 1)
    y = lax.bitcast_convert_type(i, jnp.float32)
    for _ in range(3):
        y = y * (1.5 - 0.5 * x * y * y)
    return y


def _lane_allreduce_sum(x):
    idx = lax.iota(jnp.int32, L)
    for k in (1, 2, 4, 8):
        x = x + x.at[idx ^ k].get(mode="promise_in_bounds")
    return x


def _make_sc_kernel(nb, seq):
    n_tokens = nb * seq
    tpw = n_tokens // NW
    ppw = tpw // nb
    pch = CH // nb
    nch = tpw // CH
    npair = nch // 2

    mesh = plsc.VectorSubcoreMesh(core_axis_name="c", subcore_axis_name="s")

    @functools.partial(
        pl.kernel,
        out_type=jax.ShapeDtypeStruct((n_tokens, HIDDEN), jnp.float32),
        mesh=mesh,
        scratch_types=[
            pltpu.VMEM((tpw,), jnp.int32),
            pltpu.VMEM((CH, HIDDEN), jnp.float32),
            pltpu.VMEM((CH, HIDDEN), jnp.float32),
            pltpu.VMEM((pch, HIDDEN), jnp.float32),
            pltpu.VMEM((pch, HIDDEN), jnp.float32),
            pltpu.VMEM((CH, HIDDEN), jnp.float32),
            pltpu.VMEM((CH, HIDDEN), jnp.float32),
            pltpu.VMEM((CH, HIDDEN), jnp.float32),
            pltpu.VMEM((HIDDEN,), jnp.float32),
            pltpu.VMEM((HIDDEN,), jnp.float32),
            pltpu.SemaphoreType.DMA,
            pltpu.SemaphoreType.DMA,
            pltpu.SemaphoreType.DMA,
            pltpu.SemaphoreType.DMA,
            pltpu.SemaphoreType.DMA,
            pltpu.SemaphoreType.DMA,
        ],
    )
    def k(ids_hbm, wemb_hbm, pemb_hbm, gamma_hbm, beta_hbm, out_hbm,
          idx_v, rows0, rows1, pos0, pos1, ost0, ost1, xbuf, g_v, b_v,
          gs0, gs1, ps0, ps1, ss0, ss1):
        cid = lax.axis_index("c")
        sid = lax.axis_index("s")
        wid = sid * NC + cid
        base = wid * tpw
        pos_base = wid * ppw
        pltpu.sync_copy(ids_hbm.at[pl.ds(base, tpw)], idx_v)

        rows = (rows0, rows1)
        pos = (pos0, pos1)
        ost = (ost0, ost1)
        gs = (gs0, gs1)
        ps = (ps0, ps1)
        ss = (ss0, ss1)

        def issue(c, b):
            pltpu.async_copy(
                pemb_hbm.at[pl.ds(pos_base + c * pch, pch)], pos[b], ps[b])
            pltpu.async_copy(
                wemb_hbm.at[idx_v.at[pl.ds(c * CH, CH)]], rows[b], gs[b])

        def wait_in(b):
            pltpu.make_async_copy(
                pemb_hbm.at[pl.ds(0, pch)], pos[b], ps[b]).wait()
            pltpu.make_async_copy(
                pemb_hbm.at[pl.ds(0, CH)], rows[b], gs[b]).wait()

        def wait_store(b):
            for bi in range(nb):
                pltpu.make_async_copy(
                    ost[b].at[pl.ds(bi * pch, pch)],
                    out_hbm.at[pl.ds(0, pch)], ss[b]).wait()

        def compute(c, b):
            rv, pv, ov = rows[b], pos[b], ost[b]

            def p1(j, carry):
                accs, ws, ps_ = carry
                sl = pl.ds(j * L, L)
                out = []
                for t in range(CH):
                    x = ws[t] + ps_[t // nb]
                    xbuf[t, sl] = x
                    out.append(accs[2 * t] + x)
                    out.append(accs[2 * t + 1] + x * x)
                jn = (j + 1) & (NSL - 1)
                sln = pl.ds(jn * L, L)
                nws = tuple(rv[t, sln] for t in range(CH))
                nps = tuple(pv[p_, sln] for p_ in range(pch))
                return (tuple(out), nws, nps)

            zero = jnp.zeros((L,), jnp.float32)
            sl0 = pl.ds(0, L)
            w0 = tuple(rv[t, sl0] for t in range(CH))
            p0 = tuple(pv[p_, sl0] for p_ in range(pch))
            st, _, _ = plsc.parallel_loop(
                0, NSL, carry=((zero,) * (2 * CH), w0, p0))(p1)

            stats = []
            for t in range(CH):
                mean = _lane_allreduce_sum(st[2 * t]) * (1.0 / HIDDEN)
                var = (_lane_allreduce_sum(st[2 * t + 1]) * (1.0 / HIDDEN)
                       - mean * mean)
                stats.append((mean, _rsqrt_newton(var + EPS)))

            def p2(j):
                sl = pl.ds(j * L, L)
                g = g_v[sl]
                b_ = b_v[sl]
                for t in range(CH):
                    mean, inv = stats[t]
                    ov[(t % nb) * pch + t // nb, sl] = (
                        (xbuf[t, sl] - mean) * inv * g + b_)

            plsc.parallel_loop(0, NSL, unroll=2)(p2)
            for bi in range(nb):
                pltpu.async_copy(
                    ov.at[pl.ds(bi * pch, pch)],
                    out_hbm.at[pl.ds(bi * seq + pos_base + c * pch, pch)],
                    ss[b])

        issue(0, 0)
        issue(1, 1)
        pltpu.async_copy(gamma_hbm, g_v, ss0)
        pltpu.async_copy(beta_hbm, b_v, ss1)
        pltpu.make_async_copy(gamma_hbm, g_v, ss0).wait()
        pltpu.make_async_copy(beta_hbm, b_v, ss1).wait()

        def pair(i, _):
            c0 = 2 * i
            for b in range(2):
                c = c0 + b
                wait_in(b)

                @pl.when(c >= 2)
                def _():
                    wait_store(b)

                compute(c, b)

                @pl.when(c + 2 < nch)
                def _():
                    issue(c + 2, b)
            return 0

        lax.fori_loop(0, npair, pair, 0)
        wait_store(0)
        wait_store(1)

    return k


def kernel(input_ids, word_emb, pos_emb, gamma, beta):
    b, s = input_ids.shape
    n = b * s
    ids = input_ids.T.reshape(n).astype(jnp.int32)
    k = _make_sc_kernel(b, s)
    out = k(ids, word_emb, pos_emb, gamma, beta)
    return out.reshape(b, s, HIDDEN)
